# baseline (speedup 1.0000x reference)
import functools
import math

import jax
import jax.numpy as jnp
from jax import lax
from jax.experimental import pallas as pl
from jax.experimental.pallas import tpu as pltpu
from jax.experimental.pallas import tpu_sc as plsc

_L = 2
_D = 256
_E = 8
_HD = 1024
_H = 8
_P = 16
_IMG = 224
_B = 8
_S = (_IMG // _P) ** 2 + 1
_NC = 1000
_T = _B * _S
_DH = _D // _H

_TPAD = 1664
_CH = 104
_APAD = 2 * _TPAD
_NT = 32
_TB = 128
_DROWS = _NT * _TB + _TB
_TRASH = _NT * _TB
_NSC = 32


def _gelu(x):
    p0, p1, p2, p3 = (1.128381536146987, 0.1532354653121249,
                      0.04332742502107988, 0.0007576158168315916)
    q1, q2, q3 = (0.4691895897377689, 0.09460805147350791,
                  0.00935864267969743)
    ax = jnp.abs(x)
    z = jnp.minimum(ax * (1.0 / math.sqrt(2.0)), 3.25)
    u = z * z
    pn = ((p3 * u + p2) * u + p1) * u + p0
    qn = ((q3 * u + q2) * u + q1) * u + 1.0
    erf_abs = z * pn / qn
    return 0.5 * x + 0.5 * ax * erf_abs


def _ln(x, w, b):
    m = jnp.mean(x, axis=-1, keepdims=True)
    c = x - m
    v = jnp.mean(c * c, axis=-1, keepdims=True)
    return c * jax.lax.rsqrt(v + 1e-5) * w + b


def _patch_kernel(patches_ref, w_ref, b_ref, cls_ref, pos_ref, out_ref):
    pe = jnp.dot(patches_ref[0], w_ref[...],
                 preferred_element_type=jnp.float32) + b_ref[...]
    out_ref[0, :1, :] = cls_ref[...] + pos_ref[:1, :]
    out_ref[0, 1:, :] = pe + pos_ref[1:, :]


def _patch_embed(patches, w_t, b, cls, pos):
    return pl.pallas_call(
        _patch_kernel,
        grid=(_B,),
        in_specs=[
            pl.BlockSpec((1, _S - 1, 3 * _P * _P), lambda i: (i, 0, 0)),
            pl.BlockSpec((3 * _P * _P, _D), lambda i: (0, 0)),
            pl.BlockSpec((1, _D), lambda i: (0, 0)),
            pl.BlockSpec((1, _D), lambda i: (0, 0)),
            pl.BlockSpec((_S, _D), lambda i: (0, 0)),
        ],
        out_specs=pl.BlockSpec((1, _S, _D), lambda i: (i, 0, 0)),
        out_shape=jax.ShapeDtypeStruct((_B, _S, _D), jnp.float32),
    )(patches, w_t, b, cls, pos)


def _attn_kernel(h_ref, n1w_ref, n1b_ref, wqkv_ref, bqkv_ref, wout_ref,
                 bout_ref, out_ref):
    x = h_ref[0]
    x2 = _ln(x, n1w_ref[...], n1b_ref[...])
    qkv = jnp.dot(x2, wqkv_ref[...],
                  preferred_element_type=jnp.float32) + bqkv_ref[...]
    scale = 1.0 / math.sqrt(float(_DH))
    outs = []
    for hh in range(_H):
        q = qkv[:, hh * _DH:(hh + 1) * _DH]
        k = qkv[:, _D + hh * _DH:_D + (hh + 1) * _DH]
        v = qkv[:, 2 * _D + hh * _DH:2 * _D + (hh + 1) * _DH]
        s = lax.dot_general(q, k, (((1,), (1,)), ((), ())),
                            preferred_element_type=jnp.float32) * scale
        m = jnp.max(s, axis=-1, keepdims=True)
        p = jnp.exp(s - m)
        p = p * jax.lax.reciprocal(jnp.sum(p, axis=-1, keepdims=True))
        outs.append(jnp.dot(p, v, preferred_element_type=jnp.float32))
    o = jnp.concatenate(outs, axis=1)
    proj = jnp.dot(o, wout_ref[...],
                   preferred_element_type=jnp.float32) + bout_ref[...]
    out_ref[0] = x + proj


def _attn_block(h, n1w, n1b, wqkv_t, bqkv, wout_t, bout):
    return pl.pallas_call(
        _attn_kernel,
        grid=(_B,),
        in_specs=[
            pl.BlockSpec((1, _S, _D), lambda i: (i, 0, 0)),
            pl.BlockSpec((1, _D), lambda i: (0, 0)),
            pl.BlockSpec((1, _D), lambda i: (0, 0)),
            pl.BlockSpec((_D, 3 * _D), lambda i: (0, 0)),
            pl.BlockSpec((1, 3 * _D), lambda i: (0, 0)),
            pl.BlockSpec((_D, _D), lambda i: (0, 0)),
            pl.BlockSpec((1, _D), lambda i: (0, 0)),
        ],
        out_specs=pl.BlockSpec((1, _S, _D), lambda i: (i, 0, 0)),
        out_shape=jax.ShapeDtypeStruct((_B, _S, _D), jnp.float32),
    )(h, n1w, n1b, wqkv_t, bqkv, wout_t, bout)


def _cumsum_rows(x, n):
    s = 1
    while s < n:
        shifted = jnp.concatenate(
            [jnp.zeros((s, x.shape[1]), x.dtype), x[:n - s]], axis=0)
        x = x + shifted
        s *= 2
    return x


def _route_kernel(h_ref, n2w_ref, n2b_ref, rw_ref, rb_ref,
                  x3_ref, gates_ref, slots_ref, counts_ref):
    x3 = _ln(h_ref[...], n2w_ref[...], n2b_ref[...])
    x3_ref[:_T, :] = x3
    x3_ref[_T:, :] = jnp.zeros((_TPAD - _T, _D), jnp.float32)

    logits = jnp.dot(x3, rw_ref[...],
                     preferred_element_type=jnp.float32) + rb_ref[...]
    lm = jnp.max(logits, axis=-1, keepdims=True)
    pe_ = jnp.exp(logits - lm)
    probs = pe_ * jax.lax.reciprocal(jnp.sum(pe_, axis=-1, keepdims=True))

    idx = lax.broadcasted_iota(jnp.int32, (_T, _E), 1)
    m1 = jnp.max(probs, axis=-1, keepdims=True)
    e1 = jnp.min(jnp.where(probs == m1, idx, _E), axis=-1, keepdims=True)
    oh1 = (idx == e1).astype(jnp.float32)
    probs2 = jnp.where(idx == e1, -jnp.inf, probs)
    m2 = jnp.max(probs2, axis=-1, keepdims=True)
    e2 = jnp.min(jnp.where(probs2 == m2, idx, _E), axis=-1, keepdims=True)
    oh2 = (idx == e2).astype(jnp.float32)

    gsum = m1 + m2
    g1 = m1 / gsum
    g2 = m2 / gsum
    sel0 = (idx == 0).astype(jnp.float32)
    sel1 = (idx == 1).astype(jnp.float32)
    gates_ref[...] = g1 * sel0 + g2 * sel1

    cum1 = _cumsum_rows(oh1, _T)
    c1 = cum1[_T - 1:_T, :]
    pos1 = cum1 - oh1
    cum2 = _cumsum_rows(oh2, _T)
    c2 = cum2[_T - 1:_T, :]
    pos2 = cum2 - oh2 + c1
    counts = c1 + c2

    asz = jnp.floor((counts + float(_TB - 1)) * (1.0 / _TB)) * float(_TB)
    acc = asz
    s = 1
    while s < _E:
        acc = acc + jnp.concatenate(
            [jnp.zeros((1, s), jnp.float32), acc[:, :_E - s]], axis=1)
        s *= 2
    seg = acc - asz

    slot1 = jnp.sum((pos1 + seg) * oh1, axis=-1, keepdims=True)
    slot2 = jnp.sum((pos2 + seg) * oh2, axis=-1, keepdims=True)
    slots_ref[...] = (slot1 * sel0 + slot2 * sel1).astype(jnp.int32)
    counts_ref[...] = counts.astype(jnp.int32)


def _route(hflat, n2w, n2b, rw_t, rb):
    return pl.pallas_call(
        _route_kernel,
        in_specs=[
            pl.BlockSpec((_T, _D), lambda: (0, 0)),
            pl.BlockSpec((1, _D), lambda: (0, 0)),
            pl.BlockSpec((1, _D), lambda: (0, 0)),
            pl.BlockSpec((_D, _E), lambda: (0, 0)),
            pl.BlockSpec((1, _E), lambda: (0, 0)),
        ],
        out_specs=[
            pl.BlockSpec((_TPAD, _D), lambda: (0, 0)),
            pl.BlockSpec((_T, _E), lambda: (0, 0)),
            pl.BlockSpec((_T, _E), lambda: (0, 0)),
            pl.BlockSpec((1, _E), lambda: (0, 0)),
        ],
        out_shape=[
            jax.ShapeDtypeStruct((_TPAD, _D), jnp.float32),
            jax.ShapeDtypeStruct((_T, _E), jnp.float32),
            jax.ShapeDtypeStruct((_T, _E), jnp.int32),
            jax.ShapeDtypeStruct((1, _E), jnp.int32),
        ],
    )(hflat, n2w, n2b, rw_t, rb)


def _sc_mesh():
    return plsc.VectorSubcoreMesh(core_axis_name="c", subcore_axis_name="s")


def _sc_dispatch(x3p, slots_pad):
    @functools.partial(
        pl.kernel,
        out_type=jax.ShapeDtypeStruct((_DROWS, _D), jnp.float32),
        mesh=_sc_mesh(),
        scratch_types=[
            pltpu.VMEM((_CH,), jnp.int32),
            pltpu.VMEM((_CH, _D), jnp.float32),
            pltpu.SemaphoreType.DMA,
        ],
    )
    def k(x3_hbm, slots_hbm, out_hbm, idx_v, rows_v, sem):
        w = lax.axis_index("s") * 2 + lax.axis_index("c")
        base = w * _CH
        t0 = (w % 16) * _CH
        pltpu.sync_copy(slots_hbm.at[pl.ds(base, _CH)], idx_v)
        pltpu.sync_copy(x3_hbm.at[pl.ds(t0, _CH)], rows_v)
        pltpu.async_copy(rows_v, out_hbm.at[idx_v], sem).wait()

    return k(x3p, slots_pad)


def _sc_collect(ybuf, slots_pad):
    @functools.partial(
        pl.kernel,
        out_type=jax.ShapeDtypeStruct((_APAD, _D), jnp.float32),
        mesh=_sc_mesh(),
        scratch_types=[
            pltpu.VMEM((_CH,), jnp.int32),
            pltpu.VMEM((_CH, _D), jnp.float32),
            pltpu.SemaphoreType.DMA,
        ],
    )
    def k(ybuf_hbm, slots_hbm, out_hbm, idx_v, rows_v, sem):
        w = lax.axis_index("s") * 2 + lax.axis_index("c")
        base = w * _CH
        pltpu.sync_copy(slots_hbm.at[pl.ds(base, _CH)], idx_v)
        pltpu.async_copy(ybuf_hbm.at[idx_v], rows_v, sem).wait()
        pltpu.sync_copy(rows_v, out_hbm.at[pl.ds(base, _CH)])

    return k(ybuf, slots_pad)


def _expert_kernel(te_ref, tr_ref, x_ref, w1_ref, b1_ref, w2_ref, b2_ref,
                   out_ref):
    del te_ref, tr_ref
    h1 = _gelu(jnp.dot(x_ref[...], w1_ref[0],
                       preferred_element_type=jnp.float32) + b1_ref[0])
    out_ref[...] = jnp.dot(h1, w2_ref[0],
                           preferred_element_type=jnp.float32) + b2_ref[0]


def _expert_ffn(te, tr, xdisp, w1, b1, w2, b2):
    spec = pltpu.PrefetchScalarGridSpec(
        num_scalar_prefetch=2,
        grid=(_NT,),
        in_specs=[
            pl.BlockSpec((_TB, _D), lambda j, te, tr: (tr[j], 0)),
            pl.BlockSpec((1, _D, _HD), lambda j, te, tr: (te[j], 0, 0)),
            pl.BlockSpec((1, 1, _HD), lambda j, te, tr: (te[j], 0, 0)),
            pl.BlockSpec((1, _HD, _D), lambda j, te, tr: (te[j], 0, 0)),
            pl.BlockSpec((1, 1, _D), lambda j, te, tr: (te[j], 0, 0)),
        ],
        out_specs=pl.BlockSpec((_TB, _D), lambda j, te, tr: (tr[j], 0)),
    )
    return pl.pallas_call(
        _expert_kernel,
        grid_spec=spec,
        out_shape=jax.ShapeDtypeStruct((_DROWS, _D), jnp.float32),
        compiler_params=pltpu.CompilerParams(
            dimension_semantics=("arbitrary",)),
    )(te, tr, xdisp, w1, b1, w2, b2)


def _combine_kernel(h_ref, g_ref, y_ref, out_ref):
    g1 = jnp.sum(g_ref[...] *
                 (lax.broadcasted_iota(jnp.int32, (_T, _E), 1) == 0),
                 axis=-1, keepdims=True)
    g2 = jnp.sum(g_ref[...] *
                 (lax.broadcasted_iota(jnp.int32, (_T, _E), 1) == 1),
                 axis=-1, keepdims=True)
    out_ref[...] = (h_ref[...] + g1 * y_ref[0, :_T, :]
                    + g2 * y_ref[1, :_T, :])


def _combine(hflat, gsel, gathered):
    return pl.pallas_call(
        _combine_kernel,
        in_specs=[
            pl.BlockSpec((_T, _D), lambda: (0, 0)),
            pl.BlockSpec((_T, _E), lambda: (0, 0)),
            pl.BlockSpec((2, _TPAD, _D), lambda: (0, 0, 0)),
        ],
        out_specs=pl.BlockSpec((_T, _D), lambda: (0, 0)),
        out_shape=jax.ShapeDtypeStruct((_T, _D), jnp.float32),
    )(hflat, gsel, gathered.reshape(2, _TPAD, _D))


def _head_kernel(cls_ref, fw_ref, fb_ref, w1_ref, b1_ref, w2_ref, b2_ref,
                 out_ref):
    c = _ln(cls_ref[...], fw_ref[...], fb_ref[...])
    z = _gelu(jnp.dot(c, w1_ref[...],
                      preferred_element_type=jnp.float32) + b1_ref[...])
    out_ref[...] = jnp.dot(z, w2_ref[...],
                           preferred_element_type=jnp.float32) + b2_ref[...]


def _head(cls_rows, fw, fb, h1w_t, h1b, h2w_t, h2b):
    return pl.pallas_call(
        _head_kernel,
        in_specs=[
            pl.BlockSpec((_B, _D), lambda: (0, 0)),
            pl.BlockSpec((1, _D), lambda: (0, 0)),
            pl.BlockSpec((1, _D), lambda: (0, 0)),
            pl.BlockSpec((_D, _D), lambda: (0, 0)),
            pl.BlockSpec((1, _D), lambda: (0, 0)),
            pl.BlockSpec((_D, _NC), lambda: (0, 0)),
            pl.BlockSpec((1, _NC), lambda: (0, 0)),
        ],
        out_specs=pl.BlockSpec((_B, _NC), lambda: (0, 0)),
        out_shape=jax.ShapeDtypeStruct((_B, _NC), jnp.float32),
    )(cls_rows, fw, fb, h1w_t, h1b, h2w_t, h2b)


def _moe_layer(hflat, n2w, n2b, rw_t, rb, w1, b1, w2, b2):
    x3p, gsel, slots, counts = _route(hflat, n2w, n2b, rw_t, rb)

    c = counts[0]
    ntiles = (c + (_TB - 1)) // _TB
    cumt = jnp.concatenate([jnp.zeros((1,), jnp.int32),
                            jnp.cumsum(ntiles).astype(jnp.int32)])
    total = cumt[_E]
    j = jnp.arange(_NT, dtype=jnp.int32)
    te_raw = jnp.sum((j[:, None] >= cumt[1:][None, :]).astype(jnp.int32),
                     axis=1)
    last = jnp.maximum(total - 1, 0)
    te = jnp.where(j < total, te_raw, te_raw[last])
    tr = jnp.where(j < total, j, last)

    pad = jnp.full((_TPAD - _T,), _TRASH, jnp.int32)
    slots_pad = jnp.concatenate(
        [slots[:, 0], pad, slots[:, 1], pad])

    xdisp = _sc_dispatch(x3p, slots_pad)
    ybuf = _expert_ffn(te, tr, xdisp, w1, b1, w2, b2)
    gathered = _sc_collect(ybuf, slots_pad)
    return _combine(hflat, gsel, gathered)


def kernel(x, patch_w, patch_b, cls_token, pos_embed, norm1_w, norm1_b,
           attn_in_w, attn_in_b, attn_out_w, attn_out_b, norm2_w, norm2_b,
           router_w, router_b, e_w1, e_b1, e_w2, e_b2, fnorm_w, fnorm_b,
           head1_w, head1_b, head2_w, head2_b):
    nP = _IMG // _P
    patches = x.reshape(_B, 3, nP, _P, nP, _P)
    patches = patches.transpose(0, 2, 4, 1, 3, 5).reshape(_B, nP * nP,
                                                          3 * _P * _P)
    pw_t = patch_w.reshape(_D, 3 * _P * _P).T

    h = _patch_embed(patches, pw_t, patch_b.reshape(1, _D),
                     cls_token.reshape(1, _D), pos_embed.reshape(_S, _D))

    for i in range(_L):
        h = _attn_block(
            h,
            norm1_w[i].reshape(1, _D), norm1_b[i].reshape(1, _D),
            attn_in_w[i].T, attn_in_b[i].reshape(1, 3 * _D),
            attn_out_w[i].T, attn_out_b[i].reshape(1, _D),
        )
        hflat = _moe_layer(
            h.reshape(_T, _D),
            norm2_w[i].reshape(1, _D), norm2_b[i].reshape(1, _D),
            router_w[i].T, router_b[i].reshape(1, _E),
            e_w1[i], e_b1[i].reshape(_E, 1, _HD),
            e_w2[i], e_b2[i].reshape(_E, 1, _D),
        )
        h = hflat.reshape(_B, _S, _D)

    cls_rows = h[:, 0, :]
    return _head(cls_rows, fnorm_w.reshape(1, _D), fnorm_b.reshape(1, _D),
                 head1_w.T, head1_b.reshape(1, _D),
                 head2_w.T, head2_b.reshape(1, _NC))


# speedup vs baseline: 1.0800x; 1.0800x over previous
import functools
import math

import jax
import jax.numpy as jnp
from jax import lax
from jax.experimental import pallas as pl
from jax.experimental.pallas import tpu as pltpu
from jax.experimental.pallas import tpu_sc as plsc

_L = 2
_D = 256
_E = 8
_HD = 1024
_H = 8
_P = 16
_IMG = 224
_B = 8
_S = (_IMG // _P) ** 2 + 1
_NC = 1000
_T = _B * _S
_DH = _D // _H

_TPAD = 1664
_CH = 104
_APAD = 2 * _TPAD
_NT = 32
_TB = 128
_DROWS = _NT * _TB + _TB
_TRASH = _NT * _TB
_NSC = 32


def _gelu(x):
    p0, p1, p2, p3 = (1.128381536146987, 0.1532354653121249,
                      0.04332742502107988, 0.0007576158168315916)
    q1, q2, q3 = (0.4691895897377689, 0.09460805147350791,
                  0.00935864267969743)
    ax = jnp.abs(x)
    z = jnp.minimum(ax * (1.0 / math.sqrt(2.0)), 3.25)
    u = z * z
    pn = ((p3 * u + p2) * u + p1) * u + p0
    qn = ((q3 * u + q2) * u + q1) * u + 1.0
    erf_abs = z * pn / qn
    return 0.5 * x + 0.5 * ax * erf_abs


def _ln(x, w, b):
    m = jnp.mean(x, axis=-1, keepdims=True)
    c = x - m
    v = jnp.mean(c * c, axis=-1, keepdims=True)
    return c * jax.lax.rsqrt(v + 1e-5) * w + b


_NPR = _IMG // _P


def _patch_kernel(patches_ref, w_ref, b_ref, cls_ref, pos_ref, out_ref):
    pe = lax.dot_general(patches_ref[0], w_ref[...],
                         (((1,), (1,)), ((), ())),
                         preferred_element_type=jnp.float32) + b_ref[...]
    out_ref[0, :1, :] = cls_ref[...] + pos_ref[:1, :]
    out_ref[0, 1:, :] = pe + pos_ref[1:, :]


def _patch_embed(patches, w, b, cls, pos):
    return pl.pallas_call(
        _patch_kernel,
        grid=(_B,),
        in_specs=[
            pl.BlockSpec((1, _S - 1, 3 * _P * _P), lambda i: (i, 0, 0)),
            pl.BlockSpec((_D, 3 * _P * _P), lambda i: (0, 0)),
            pl.BlockSpec((1, _D), lambda i: (0, 0)),
            pl.BlockSpec((1, _D), lambda i: (0, 0)),
            pl.BlockSpec((_S, _D), lambda i: (0, 0)),
        ],
        out_specs=pl.BlockSpec((1, _S, _D), lambda i: (i, 0, 0)),
        out_shape=jax.ShapeDtypeStruct((_B, _S, _D), jnp.float32),
    )(patches, w, b, cls, pos)


def _attn_kernel(h_ref, n1w_ref, n1b_ref, wqkv_ref, bqkv_ref, wout_ref,
                 bout_ref, out_ref):
    x = h_ref[0]
    x2 = _ln(x, n1w_ref[0], n1b_ref[0])
    qkv = lax.dot_general(x2, wqkv_ref[0], (((1,), (1,)), ((), ())),
                          preferred_element_type=jnp.float32) + bqkv_ref[0]
    scale = 1.0 / math.sqrt(float(_DH))
    outs = []
    for hh in range(_H):
        q = qkv[:, hh * _DH:(hh + 1) * _DH]
        k = qkv[:, _D + hh * _DH:_D + (hh + 1) * _DH]
        v = qkv[:, 2 * _D + hh * _DH:2 * _D + (hh + 1) * _DH]
        s = lax.dot_general(q, k, (((1,), (1,)), ((), ())),
                            preferred_element_type=jnp.float32) * scale
        m = jnp.max(s, axis=-1, keepdims=True)
        p = jnp.exp(s - m)
        p = p * jax.lax.reciprocal(jnp.sum(p, axis=-1, keepdims=True))
        outs.append(jnp.dot(p, v, preferred_element_type=jnp.float32))
    o = jnp.concatenate(outs, axis=1)
    proj = lax.dot_general(o, wout_ref[0], (((1,), (1,)), ((), ())),
                           preferred_element_type=jnp.float32) + bout_ref[0]
    out_ref[0] = x + proj


def _attn_block(li, h, n1w, n1b, wqkv, bqkv, wout, bout):
    return pl.pallas_call(
        _attn_kernel,
        grid=(_B,),
        in_specs=[
            pl.BlockSpec((1, _S, _D), lambda i: (i, 0, 0)),
            pl.BlockSpec((1, 1, _D), lambda i, li=li: (li, 0, 0)),
            pl.BlockSpec((1, 1, _D), lambda i, li=li: (li, 0, 0)),
            pl.BlockSpec((1, 3 * _D, _D), lambda i, li=li: (li, 0, 0)),
            pl.BlockSpec((1, 1, 3 * _D), lambda i, li=li: (li, 0, 0)),
            pl.BlockSpec((1, _D, _D), lambda i, li=li: (li, 0, 0)),
            pl.BlockSpec((1, 1, _D), lambda i, li=li: (li, 0, 0)),
        ],
        out_specs=pl.BlockSpec((1, _S, _D), lambda i: (i, 0, 0)),
        out_shape=jax.ShapeDtypeStruct((_B, _S, _D), jnp.float32),
    )(h, n1w, n1b, wqkv, bqkv, wout, bout)


def _cumsum_rows(x, n):
    s = 1
    while s < n:
        shifted = jnp.concatenate(
            [jnp.zeros((s, x.shape[1]), x.dtype), x[:n - s]], axis=0)
        x = x + shifted
        s *= 2
    return x


def _route_kernel(h_ref, n2w_ref, n2b_ref, rw_ref, rb_ref,
                  x3_ref, gates_ref, slots_ref, te_ref, tr_ref):
    x3 = _ln(h_ref[...], n2w_ref[0], n2b_ref[0])
    x3_ref[:_T, :] = x3
    x3_ref[_T:, :] = jnp.zeros((_TPAD - _T, _D), jnp.float32)

    logits = lax.dot_general(x3, rw_ref[0], (((1,), (1,)), ((), ())),
                             preferred_element_type=jnp.float32) + rb_ref[0]
    lm = jnp.max(logits, axis=-1, keepdims=True)
    pe_ = jnp.exp(logits - lm)
    probs = pe_ * jax.lax.reciprocal(jnp.sum(pe_, axis=-1, keepdims=True))

    idx = lax.broadcasted_iota(jnp.int32, (_T, _E), 1)
    m1 = jnp.max(probs, axis=-1, keepdims=True)
    e1 = jnp.min(jnp.where(probs == m1, idx, _E), axis=-1, keepdims=True)
    oh1 = (idx == e1).astype(jnp.float32)
    probs2 = jnp.where(idx == e1, -jnp.inf, probs)
    m2 = jnp.max(probs2, axis=-1, keepdims=True)
    e2 = jnp.min(jnp.where(probs2 == m2, idx, _E), axis=-1, keepdims=True)
    oh2 = (idx == e2).astype(jnp.float32)

    gsum = m1 + m2
    g1 = m1 / gsum
    g2 = m2 / gsum
    sel0 = (idx == 0).astype(jnp.float32)
    sel1 = (idx == 1).astype(jnp.float32)
    gates_ref[...] = g1 * sel0 + g2 * sel1

    cum1 = _cumsum_rows(oh1, _T)
    c1 = cum1[_T - 1:_T, :]
    pos1 = cum1 - oh1
    cum2 = _cumsum_rows(oh2, _T)
    c2 = cum2[_T - 1:_T, :]
    pos2 = cum2 - oh2 + c1
    counts = c1 + c2

    asz = jnp.floor((counts + float(_TB - 1)) * (1.0 / _TB)) * float(_TB)
    acc = asz
    s = 1
    while s < _E:
        acc = acc + jnp.concatenate(
            [jnp.zeros((1, s), jnp.float32), acc[:, :_E - s]], axis=1)
        s *= 2
    seg = acc - asz

    slot1 = jnp.sum((pos1 + seg) * oh1, axis=-1, keepdims=True)
    slot2 = jnp.sum((pos2 + seg) * oh2, axis=-1, keepdims=True)

    trash = jnp.full((1, _TPAD - _T), float(_TRASH), jnp.float32)
    r1 = jnp.concatenate([slot1.reshape(1, _T), trash], axis=1)
    r2 = jnp.concatenate([slot2.reshape(1, _T), trash], axis=1)
    slots_ref[...] = jnp.concatenate([r1, r2], axis=0).astype(jnp.int32)

    ntiles = jnp.floor((counts + float(_TB - 1)) * (1.0 / _TB))
    cum = ntiles
    s = 1
    while s < _E:
        cum = cum + jnp.concatenate(
            [jnp.zeros((1, s), jnp.float32), cum[:, :_E - s]], axis=1)
        s *= 2
    lane8 = lax.broadcasted_iota(jnp.int32, (1, _E), 1)
    total = jnp.sum(jnp.where(lane8 == _E - 1, cum, 0.0), axis=-1,
                    keepdims=True)
    jj = lax.broadcasted_iota(jnp.int32, (_NT, _E), 0).astype(jnp.float32)
    te_raw = jnp.sum((jj >= cum).astype(jnp.float32), axis=-1,
                     keepdims=True)
    jcol = lax.broadcasted_iota(jnp.int32, (_NT, 1), 0).astype(jnp.float32)
    last = jnp.maximum(total - 1.0, 0.0)
    te_last = jnp.sum(jnp.where(jcol == last, te_raw, 0.0), axis=0,
                      keepdims=True)
    live = jcol < total
    te_ref[...] = jnp.where(live, te_raw, te_last).astype(jnp.int32)
    tr_ref[...] = jnp.where(live, jcol, last).astype(jnp.int32)


def _route(li, hflat, n2w, n2b, rw, rb):
    return pl.pallas_call(
        _route_kernel,
        grid=(1,),
        in_specs=[
            pl.BlockSpec((_T, _D), lambda g: (0, 0)),
            pl.BlockSpec((1, 1, _D), lambda g, li=li: (li, 0, 0)),
            pl.BlockSpec((1, 1, _D), lambda g, li=li: (li, 0, 0)),
            pl.BlockSpec((1, _E, _D), lambda g, li=li: (li, 0, 0)),
            pl.BlockSpec((1, 1, _E), lambda g, li=li: (li, 0, 0)),
        ],
        out_specs=[
            pl.BlockSpec((_TPAD, _D), lambda g: (0, 0)),
            pl.BlockSpec((_T, _E), lambda g: (0, 0)),
            pl.BlockSpec((2, _TPAD), lambda g: (0, 0)),
            pl.BlockSpec((_NT, 1), lambda g: (0, 0)),
            pl.BlockSpec((_NT, 1), lambda g: (0, 0)),
        ],
        out_shape=[
            jax.ShapeDtypeStruct((_TPAD, _D), jnp.float32),
            jax.ShapeDtypeStruct((_T, _E), jnp.float32),
            jax.ShapeDtypeStruct((2, _TPAD), jnp.int32),
            jax.ShapeDtypeStruct((_NT, 1), jnp.int32),
            jax.ShapeDtypeStruct((_NT, 1), jnp.int32),
        ],
    )(hflat, n2w, n2b, rw, rb)


def _sc_mesh():
    return plsc.VectorSubcoreMesh(core_axis_name="c", subcore_axis_name="s")


def _sc_dispatch(x3p, slots_pad):
    @functools.partial(
        pl.kernel,
        out_type=jax.ShapeDtypeStruct((_DROWS, _D), jnp.float32),
        mesh=_sc_mesh(),
        scratch_types=[
            pltpu.VMEM((_CH,), jnp.int32),
            pltpu.VMEM((_CH, _D), jnp.float32),
            pltpu.SemaphoreType.DMA,
        ],
    )
    def k(x3_hbm, slots_hbm, out_hbm, idx_v, rows_v, sem):
        w = lax.axis_index("s") * 2 + lax.axis_index("c")
        base = w * _CH
        t0 = (w % 16) * _CH
        pltpu.sync_copy(slots_hbm.at[pl.ds(base, _CH)], idx_v)
        pltpu.sync_copy(x3_hbm.at[pl.ds(t0, _CH)], rows_v)
        pltpu.async_copy(rows_v, out_hbm.at[idx_v], sem).wait()

    return k(x3p, slots_pad)


def _sc_collect(ybuf, slots_pad):
    @functools.partial(
        pl.kernel,
        out_type=jax.ShapeDtypeStruct((_APAD, _D), jnp.float32),
        mesh=_sc_mesh(),
        scratch_types=[
            pltpu.VMEM((_CH,), jnp.int32),
            pltpu.VMEM((_CH, _D), jnp.float32),
            pltpu.SemaphoreType.DMA,
        ],
    )
    def k(ybuf_hbm, slots_hbm, out_hbm, idx_v, rows_v, sem):
        w = lax.axis_index("s") * 2 + lax.axis_index("c")
        base = w * _CH
        pltpu.sync_copy(slots_hbm.at[pl.ds(base, _CH)], idx_v)
        pltpu.async_copy(ybuf_hbm.at[idx_v], rows_v, sem).wait()
        pltpu.sync_copy(rows_v, out_hbm.at[pl.ds(base, _CH)])

    return k(ybuf, slots_pad)


def _expert_kernel(te_ref, tr_ref, x_ref, w1_ref, b1_ref, w2_ref, b2_ref,
                   out_ref):
    del te_ref, tr_ref
    h1 = _gelu(jnp.dot(x_ref[...], w1_ref[0, 0],
                       preferred_element_type=jnp.float32) + b1_ref[0, 0])
    out_ref[...] = jnp.dot(h1, w2_ref[0, 0],
                           preferred_element_type=jnp.float32) + b2_ref[0, 0]


def _expert_ffn(li, te, tr, xdisp, w1, b1, w2, b2):
    spec = pltpu.PrefetchScalarGridSpec(
        num_scalar_prefetch=2,
        grid=(_NT,),
        in_specs=[
            pl.BlockSpec((_TB, _D), lambda j, te, tr: (tr[j], 0)),
            pl.BlockSpec((1, 1, _D, _HD),
                         lambda j, te, tr, li=li: (li, te[j], 0, 0)),
            pl.BlockSpec((1, 1, 1, _HD),
                         lambda j, te, tr, li=li: (li, te[j], 0, 0)),
            pl.BlockSpec((1, 1, _HD, _D),
                         lambda j, te, tr, li=li: (li, te[j], 0, 0)),
            pl.BlockSpec((1, 1, 1, _D),
                         lambda j, te, tr, li=li: (li, te[j], 0, 0)),
        ],
        out_specs=pl.BlockSpec((_TB, _D), lambda j, te, tr: (tr[j], 0)),
    )
    return pl.pallas_call(
        _expert_kernel,
        grid_spec=spec,
        out_shape=jax.ShapeDtypeStruct((_DROWS, _D), jnp.float32),
        compiler_params=pltpu.CompilerParams(
            dimension_semantics=("arbitrary",)),
    )(te, tr, xdisp, w1, b1, w2, b2)


def _combine_kernel(h_ref, g_ref, y_ref, out_ref):
    g1 = jnp.sum(g_ref[...] *
                 (lax.broadcasted_iota(jnp.int32, (_T, _E), 1) == 0),
                 axis=-1, keepdims=True)
    g2 = jnp.sum(g_ref[...] *
                 (lax.broadcasted_iota(jnp.int32, (_T, _E), 1) == 1),
                 axis=-1, keepdims=True)
    out_ref[...] = (h_ref[...] + g1 * y_ref[0, :_T, :]
                    + g2 * y_ref[1, :_T, :])


def _combine(hflat, gsel, gathered):
    return pl.pallas_call(
        _combine_kernel,
        in_specs=[
            pl.BlockSpec((_T, _D), lambda: (0, 0)),
            pl.BlockSpec((_T, _E), lambda: (0, 0)),
            pl.BlockSpec((2, _TPAD, _D), lambda: (0, 0, 0)),
        ],
        out_specs=pl.BlockSpec((_T, _D), lambda: (0, 0)),
        out_shape=jax.ShapeDtypeStruct((_T, _D), jnp.float32),
    )(hflat, gsel, gathered.reshape(2, _TPAD, _D))


def _head_kernel(cls_ref, fw_ref, fb_ref, w1_ref, b1_ref, w2_ref, b2_ref,
                 out_ref):
    c = _ln(cls_ref[...], fw_ref[...], fb_ref[...])
    z = _gelu(lax.dot_general(c, w1_ref[...], (((1,), (1,)), ((), ())),
                              preferred_element_type=jnp.float32)
              + b1_ref[...])
    out_ref[...] = lax.dot_general(z, w2_ref[...], (((1,), (1,)), ((), ())),
                                   preferred_element_type=jnp.float32
                                   ) + b2_ref[...]


def _head(cls_rows, fw, fb, h1w_t, h1b, h2w_t, h2b):
    return pl.pallas_call(
        _head_kernel,
        in_specs=[
            pl.BlockSpec((_B, _D), lambda: (0, 0)),
            pl.BlockSpec((1, _D), lambda: (0, 0)),
            pl.BlockSpec((1, _D), lambda: (0, 0)),
            pl.BlockSpec((_D, _D), lambda: (0, 0)),
            pl.BlockSpec((1, _D), lambda: (0, 0)),
            pl.BlockSpec((_NC, _D), lambda: (0, 0)),
            pl.BlockSpec((1, _NC), lambda: (0, 0)),
        ],
        out_specs=pl.BlockSpec((_B, _NC), lambda: (0, 0)),
        out_shape=jax.ShapeDtypeStruct((_B, _NC), jnp.float32),
    )(cls_rows, fw, fb, h1w_t, h1b, h2w_t, h2b)


def _moe_layer(li, hflat, n2w, n2b, rw, rb, w1, b1, w2, b2):
    x3p, gsel, slots2, te, tr = _route(li, hflat, n2w, n2b, rw, rb)
    slots_pad = slots2.reshape(_APAD)
    te = te.reshape(_NT)
    tr = tr.reshape(_NT)

    xdisp = _sc_dispatch(x3p, slots_pad)
    ybuf = _expert_ffn(li, te, tr, xdisp, w1, b1, w2, b2)
    gathered = _sc_collect(ybuf, slots_pad)
    return _combine(hflat, gsel, gathered)


def kernel(x, patch_w, patch_b, cls_token, pos_embed, norm1_w, norm1_b,
           attn_in_w, attn_in_b, attn_out_w, attn_out_b, norm2_w, norm2_b,
           router_w, router_b, e_w1, e_b1, e_w2, e_b2, fnorm_w, fnorm_b,
           head1_w, head1_b, head2_w, head2_b):
    patches = x.reshape(_B, 3, _NPR, _P, _NPR, _P)
    patches = patches.transpose(0, 2, 4, 1, 3, 5).reshape(
        _B, _NPR * _NPR, 3 * _P * _P)
    pw = patch_w.reshape(_D, 3 * _P * _P)

    h = _patch_embed(patches, pw, patch_b.reshape(1, _D),
                     cls_token.reshape(1, _D), pos_embed.reshape(_S, _D))

    b1r = e_b1.reshape(_L, _E, 1, _HD)
    b2r = e_b2.reshape(_L, _E, 1, _D)
    n1w = norm1_w.reshape(_L, 1, _D)
    n1b = norm1_b.reshape(_L, 1, _D)
    bqkv = attn_in_b.reshape(_L, 1, 3 * _D)
    bout = attn_out_b.reshape(_L, 1, _D)
    n2w = norm2_w.reshape(_L, 1, _D)
    n2b = norm2_b.reshape(_L, 1, _D)
    rbr = router_b.reshape(_L, 1, _E)
    for i in range(_L):
        h = _attn_block(i, h, n1w, n1b, attn_in_w, bqkv, attn_out_w, bout)
        hflat = _moe_layer(i, h.reshape(_T, _D), n2w, n2b,
                           router_w, rbr, e_w1, b1r, e_w2, b2r)
        h = hflat.reshape(_B, _S, _D)

    cls_rows = h[:, 0, :]
    return _head(cls_rows, fnorm_w.reshape(1, _D), fnorm_b.reshape(1, _D),
                 head1_w, head1_b.reshape(1, _D),
                 head2_w, head2_b.reshape(1, _NC))


# speedup vs baseline: 1.2173x; 1.1271x over previous
import functools
import math

import jax
import jax.numpy as jnp
from jax import lax
from jax.experimental import pallas as pl
from jax.experimental.pallas import tpu as pltpu
from jax.experimental.pallas import tpu_sc as plsc

_L = 2
_D = 256
_E = 8
_HD = 1024
_H = 8
_P = 16
_IMG = 224
_B = 8
_S = (_IMG // _P) ** 2 + 1
_NC = 1000
_T = _B * _S
_DH = _D // _H

_TPAD = 1664
_CH = 104
_APAD = 2 * _TPAD
_NT = 32
_TB = 128
_DROWS = _NT * _TB + _TB
_TRASH = _NT * _TB
_NSC = 32


def _gelu(x):
    p0, p1, p2, p3 = (1.128381536146987, 0.1532354653121249,
                      0.04332742502107988, 0.0007576158168315916)
    q1, q2, q3 = (0.4691895897377689, 0.09460805147350791,
                  0.00935864267969743)
    ax = jnp.abs(x)
    z = jnp.minimum(ax * (1.0 / math.sqrt(2.0)), 3.25)
    u = z * z
    pn = ((p3 * u + p2) * u + p1) * u + p0
    qn = ((q3 * u + q2) * u + q1) * u + 1.0
    erf_abs = z * pn / qn
    return 0.5 * x + 0.5 * ax * erf_abs


def _ln(x, w, b):
    m = jnp.mean(x, axis=-1, keepdims=True)
    c = x - m
    v = jnp.mean(c * c, axis=-1, keepdims=True)
    return c * jax.lax.rsqrt(v + 1e-5) * w + b


_NPR = _IMG // _P


def _patch_kernel(x_ref, w_ref, b_ref, cls_ref, pos_ref, out_ref, acc_s):
    c = pl.program_id(0)
    i = pl.program_id(1)
    xs = x_ref[:, 0, :, 0, :, :].reshape(_B * _NPR * _NPR, _P)
    contrib = jnp.dot(xs, w_ref[0, 0], preferred_element_type=jnp.float32)
    first = jnp.logical_and(c == 0, i == 0)

    @pl.when(first)
    def _init():
        acc_s[...] = contrib

    @pl.when(jnp.logical_not(first))
    def _acc():
        acc_s[...] += contrib

    @pl.when(jnp.logical_and(c == 2, i == _P - 1))
    def _fin():
        pe = (acc_s[...] + b_ref[...]).reshape(_B, _NPR * _NPR, _D)
        cls_row = cls_ref[...] + pos_ref[:1, :]
        for b in range(_B):
            out_ref[b, :1, :] = cls_row
            out_ref[b, 1:, :] = pe[b] + pos_ref[1:, :]


def _patch_embed(x6, w, b, cls, pos):
    return pl.pallas_call(
        _patch_kernel,
        grid=(3, _P),
        in_specs=[
            pl.BlockSpec((_B, 1, _NPR, 1, _NPR, _P),
                         lambda c, i: (0, c, 0, i, 0, 0)),
            pl.BlockSpec((1, 1, _P, _D), lambda c, i: (c, i, 0, 0)),
            pl.BlockSpec((1, _D), lambda c, i: (0, 0)),
            pl.BlockSpec((1, _D), lambda c, i: (0, 0)),
            pl.BlockSpec((_S, _D), lambda c, i: (0, 0)),
        ],
        out_specs=pl.BlockSpec((_B, _S, _D), lambda c, i: (0, 0, 0)),
        out_shape=jax.ShapeDtypeStruct((_B, _S, _D), jnp.float32),
        scratch_shapes=[pltpu.VMEM((_B * _NPR * _NPR, _D), jnp.float32)],
        compiler_params=pltpu.CompilerParams(
            dimension_semantics=("arbitrary", "arbitrary")),
    )(x6, w, b, cls, pos)


def _attn_kernel(h_ref, n1w_ref, n1b_ref, wqkv_ref, bqkv_ref, wout_ref,
                 bout_ref, out_ref):
    x = h_ref[0]
    x2 = _ln(x, n1w_ref[0], n1b_ref[0])
    qkv = lax.dot_general(x2, wqkv_ref[0], (((1,), (1,)), ((), ())),
                          preferred_element_type=jnp.float32) + bqkv_ref[0]
    scale = 1.0 / math.sqrt(float(_DH))
    outs = []
    for hh in range(_H):
        q = qkv[:, hh * _DH:(hh + 1) * _DH]
        k = qkv[:, _D + hh * _DH:_D + (hh + 1) * _DH]
        v = qkv[:, 2 * _D + hh * _DH:2 * _D + (hh + 1) * _DH]
        s = lax.dot_general(q, k, (((1,), (1,)), ((), ())),
                            preferred_element_type=jnp.float32) * scale
        m = jnp.max(s, axis=-1, keepdims=True)
        p = jnp.exp(s - m)
        p = p * jax.lax.reciprocal(jnp.sum(p, axis=-1, keepdims=True))
        outs.append(jnp.dot(p, v, preferred_element_type=jnp.float32))
    o = jnp.concatenate(outs, axis=1)
    proj = lax.dot_general(o, wout_ref[0], (((1,), (1,)), ((), ())),
                           preferred_element_type=jnp.float32) + bout_ref[0]
    out_ref[0] = x + proj


def _attn_block(li, h, n1w, n1b, wqkv, bqkv, wout, bout):
    return pl.pallas_call(
        _attn_kernel,
        grid=(_B,),
        in_specs=[
            pl.BlockSpec((1, _S, _D), lambda i: (i, 0, 0)),
            pl.BlockSpec((1, 1, _D), lambda i, li=li: (li, 0, 0)),
            pl.BlockSpec((1, 1, _D), lambda i, li=li: (li, 0, 0)),
            pl.BlockSpec((1, 3 * _D, _D), lambda i, li=li: (li, 0, 0)),
            pl.BlockSpec((1, 1, 3 * _D), lambda i, li=li: (li, 0, 0)),
            pl.BlockSpec((1, _D, _D), lambda i, li=li: (li, 0, 0)),
            pl.BlockSpec((1, 1, _D), lambda i, li=li: (li, 0, 0)),
        ],
        out_specs=pl.BlockSpec((1, _S, _D), lambda i: (i, 0, 0)),
        out_shape=jax.ShapeDtypeStruct((_B, _S, _D), jnp.float32),
    )(h, n1w, n1b, wqkv, bqkv, wout, bout)


def _cumsum_rows(x, n):
    s = 1
    while s < n:
        shifted = jnp.concatenate(
            [jnp.zeros((s, x.shape[1]), x.dtype), x[:n - s]], axis=0)
        x = x + shifted
        s *= 2
    return x


def _route_kernel(h_ref, n2w_ref, n2b_ref, rw_ref, rb_ref,
                  x3_ref, gates_ref, slots_ref, te_ref, tr_ref):
    x3 = _ln(h_ref[...], n2w_ref[0], n2b_ref[0])
    x3_ref[:_T, :] = x3
    x3_ref[_T:, :] = jnp.zeros((_TPAD - _T, _D), jnp.float32)

    logits = lax.dot_general(x3, rw_ref[0], (((1,), (1,)), ((), ())),
                             preferred_element_type=jnp.float32) + rb_ref[0]
    lm = jnp.max(logits, axis=-1, keepdims=True)
    pe_ = jnp.exp(logits - lm)
    probs = pe_ * jax.lax.reciprocal(jnp.sum(pe_, axis=-1, keepdims=True))

    idx = lax.broadcasted_iota(jnp.int32, (_T, _E), 1)
    m1 = jnp.max(probs, axis=-1, keepdims=True)
    e1 = jnp.min(jnp.where(probs == m1, idx, _E), axis=-1, keepdims=True)
    oh1 = (idx == e1).astype(jnp.float32)
    probs2 = jnp.where(idx == e1, -jnp.inf, probs)
    m2 = jnp.max(probs2, axis=-1, keepdims=True)
    e2 = jnp.min(jnp.where(probs2 == m2, idx, _E), axis=-1, keepdims=True)
    oh2 = (idx == e2).astype(jnp.float32)

    gsum = m1 + m2
    g1 = m1 / gsum
    g2 = m2 / gsum
    sel0 = (idx == 0).astype(jnp.float32)
    sel1 = (idx == 1).astype(jnp.float32)
    gates_ref[...] = g1 * sel0 + g2 * sel1

    cum1 = _cumsum_rows(oh1, _T)
    c1 = cum1[_T - 1:_T, :]
    pos1 = cum1 - oh1
    cum2 = _cumsum_rows(oh2, _T)
    c2 = cum2[_T - 1:_T, :]
    pos2 = cum2 - oh2 + c1
    counts = c1 + c2

    asz = jnp.floor((counts + float(_TB - 1)) * (1.0 / _TB)) * float(_TB)
    acc = asz
    s = 1
    while s < _E:
        acc = acc + jnp.concatenate(
            [jnp.zeros((1, s), jnp.float32), acc[:, :_E - s]], axis=1)
        s *= 2
    seg = acc - asz

    slot1 = jnp.sum((pos1 + seg) * oh1, axis=-1, keepdims=True)
    slot2 = jnp.sum((pos2 + seg) * oh2, axis=-1, keepdims=True)

    trash = jnp.full((1, _TPAD - _T), float(_TRASH), jnp.float32)
    r1 = jnp.concatenate([slot1.reshape(1, _T), trash], axis=1)
    r2 = jnp.concatenate([slot2.reshape(1, _T), trash], axis=1)
    slots_ref[...] = jnp.concatenate([r1, r2], axis=0).astype(jnp.int32)

    ntiles = jnp.floor((counts + float(_TB - 1)) * (1.0 / _TB))
    cum = ntiles
    s = 1
    while s < _E:
        cum = cum + jnp.concatenate(
            [jnp.zeros((1, s), jnp.float32), cum[:, :_E - s]], axis=1)
        s *= 2
    lane8 = lax.broadcasted_iota(jnp.int32, (1, _E), 1)
    total = jnp.sum(jnp.where(lane8 == _E - 1, cum, 0.0), axis=-1,
                    keepdims=True)
    jj = lax.broadcasted_iota(jnp.int32, (_NT, _E), 0).astype(jnp.float32)
    te_raw = jnp.sum((jj >= cum).astype(jnp.float32), axis=-1,
                     keepdims=True)
    jcol = lax.broadcasted_iota(jnp.int32, (_NT, 1), 0).astype(jnp.float32)
    last = jnp.maximum(total - 1.0, 0.0)
    te_last = jnp.sum(jnp.where(jcol == last, te_raw, 0.0), axis=0,
                      keepdims=True)
    live = jcol < total
    te_ref[...] = jnp.where(live, te_raw, te_last).astype(jnp.int32)
    tr_ref[...] = jnp.where(live, jcol, last).astype(jnp.int32)


def _route(li, hflat, n2w, n2b, rw, rb):
    return pl.pallas_call(
        _route_kernel,
        grid=(1,),
        in_specs=[
            pl.BlockSpec((_T, _D), lambda g: (0, 0)),
            pl.BlockSpec((1, 1, _D), lambda g, li=li: (li, 0, 0)),
            pl.BlockSpec((1, 1, _D), lambda g, li=li: (li, 0, 0)),
            pl.BlockSpec((1, _E, _D), lambda g, li=li: (li, 0, 0)),
            pl.BlockSpec((1, 1, _E), lambda g, li=li: (li, 0, 0)),
        ],
        out_specs=[
            pl.BlockSpec((_TPAD, _D), lambda g: (0, 0)),
            pl.BlockSpec((_T, _E), lambda g: (0, 0)),
            pl.BlockSpec((2, _TPAD), lambda g: (0, 0)),
            pl.BlockSpec((_NT, 1), lambda g: (0, 0)),
            pl.BlockSpec((_NT, 1), lambda g: (0, 0)),
        ],
        out_shape=[
            jax.ShapeDtypeStruct((_TPAD, _D), jnp.float32),
            jax.ShapeDtypeStruct((_T, _E), jnp.float32),
            jax.ShapeDtypeStruct((2, _TPAD), jnp.int32),
            jax.ShapeDtypeStruct((_NT, 1), jnp.int32),
            jax.ShapeDtypeStruct((_NT, 1), jnp.int32),
        ],
    )(hflat, n2w, n2b, rw, rb)


def _sc_mesh():
    return plsc.VectorSubcoreMesh(core_axis_name="c", subcore_axis_name="s")


def _sc_dispatch(x3p, slots_pad):
    @functools.partial(
        pl.kernel,
        out_type=jax.ShapeDtypeStruct((_DROWS, _D), jnp.float32),
        mesh=_sc_mesh(),
        scratch_types=[
            pltpu.VMEM((_CH,), jnp.int32),
            pltpu.VMEM((_CH, _D), jnp.float32),
            pltpu.SemaphoreType.DMA,
        ],
    )
    def k(x3_hbm, slots_hbm, out_hbm, idx_v, rows_v, sem):
        w = lax.axis_index("s") * 2 + lax.axis_index("c")
        base = w * _CH
        t0 = (w % 16) * _CH
        pltpu.sync_copy(slots_hbm.at[pl.ds(base, _CH)], idx_v)
        pltpu.sync_copy(x3_hbm.at[pl.ds(t0, _CH)], rows_v)
        pltpu.async_copy(rows_v, out_hbm.at[idx_v], sem).wait()

    return k(x3p, slots_pad)


def _sc_collect(ybuf, slots_pad):
    @functools.partial(
        pl.kernel,
        out_type=jax.ShapeDtypeStruct((_APAD, _D), jnp.float32),
        mesh=_sc_mesh(),
        scratch_types=[
            pltpu.VMEM((_CH,), jnp.int32),
            pltpu.VMEM((_CH, _D), jnp.float32),
            pltpu.SemaphoreType.DMA,
        ],
    )
    def k(ybuf_hbm, slots_hbm, out_hbm, idx_v, rows_v, sem):
        w = lax.axis_index("s") * 2 + lax.axis_index("c")
        base = w * _CH
        pltpu.sync_copy(slots_hbm.at[pl.ds(base, _CH)], idx_v)
        pltpu.async_copy(ybuf_hbm.at[idx_v], rows_v, sem).wait()
        pltpu.sync_copy(rows_v, out_hbm.at[pl.ds(base, _CH)])

    return k(ybuf, slots_pad)


def _expert_kernel(te_ref, tr_ref, x_ref, w1_ref, b1_ref, w2_ref, b2_ref,
                   out_ref):
    del te_ref, tr_ref
    h1 = _gelu(jnp.dot(x_ref[...], w1_ref[0, 0],
                       preferred_element_type=jnp.float32) + b1_ref[0, 0])
    out_ref[...] = jnp.dot(h1, w2_ref[0, 0],
                           preferred_element_type=jnp.float32) + b2_ref[0, 0]


def _expert_ffn(li, te, tr, xdisp, w1, b1, w2, b2):
    spec = pltpu.PrefetchScalarGridSpec(
        num_scalar_prefetch=2,
        grid=(_NT,),
        in_specs=[
            pl.BlockSpec((_TB, _D), lambda j, te, tr: (tr[j], 0)),
            pl.BlockSpec((1, 1, _D, _HD),
                         lambda j, te, tr, li=li: (li, te[j], 0, 0)),
            pl.BlockSpec((1, 1, 1, _HD),
                         lambda j, te, tr, li=li: (li, te[j], 0, 0)),
            pl.BlockSpec((1, 1, _HD, _D),
                         lambda j, te, tr, li=li: (li, te[j], 0, 0)),
            pl.BlockSpec((1, 1, 1, _D),
                         lambda j, te, tr, li=li: (li, te[j], 0, 0)),
        ],
        out_specs=pl.BlockSpec((_TB, _D), lambda j, te, tr: (tr[j], 0)),
    )
    return pl.pallas_call(
        _expert_kernel,
        grid_spec=spec,
        out_shape=jax.ShapeDtypeStruct((_DROWS, _D), jnp.float32),
        compiler_params=pltpu.CompilerParams(
            dimension_semantics=("arbitrary",)),
    )(te, tr, xdisp, w1, b1, w2, b2)


def _combine_kernel(h_ref, g_ref, y_ref, out_ref):
    g1 = jnp.sum(g_ref[...] *
                 (lax.broadcasted_iota(jnp.int32, (_T, _E), 1) == 0),
                 axis=-1, keepdims=True)
    g2 = jnp.sum(g_ref[...] *
                 (lax.broadcasted_iota(jnp.int32, (_T, _E), 1) == 1),
                 axis=-1, keepdims=True)
    out_ref[...] = (h_ref[...] + g1 * y_ref[0, :_T, :]
                    + g2 * y_ref[1, :_T, :])


def _combine(hflat, gsel, gathered):
    return pl.pallas_call(
        _combine_kernel,
        in_specs=[
            pl.BlockSpec((_T, _D), lambda: (0, 0)),
            pl.BlockSpec((_T, _E), lambda: (0, 0)),
            pl.BlockSpec((2, _TPAD, _D), lambda: (0, 0, 0)),
        ],
        out_specs=pl.BlockSpec((_T, _D), lambda: (0, 0)),
        out_shape=jax.ShapeDtypeStruct((_T, _D), jnp.float32),
    )(hflat, gsel, gathered.reshape(2, _TPAD, _D))


def _head_kernel(cls_ref, fw_ref, fb_ref, w1_ref, b1_ref, w2_ref, b2_ref,
                 out_ref):
    c = _ln(cls_ref[...], fw_ref[...], fb_ref[...])
    z = _gelu(lax.dot_general(c, w1_ref[...], (((1,), (1,)), ((), ())),
                              preferred_element_type=jnp.float32)
              + b1_ref[...])
    out_ref[...] = lax.dot_general(z, w2_ref[...], (((1,), (1,)), ((), ())),
                                   preferred_element_type=jnp.float32
                                   ) + b2_ref[...]


def _head(cls_rows, fw, fb, h1w_t, h1b, h2w_t, h2b):
    return pl.pallas_call(
        _head_kernel,
        in_specs=[
            pl.BlockSpec((_B, _D), lambda: (0, 0)),
            pl.BlockSpec((1, _D), lambda: (0, 0)),
            pl.BlockSpec((1, _D), lambda: (0, 0)),
            pl.BlockSpec((_D, _D), lambda: (0, 0)),
            pl.BlockSpec((1, _D), lambda: (0, 0)),
            pl.BlockSpec((_NC, _D), lambda: (0, 0)),
            pl.BlockSpec((1, _NC), lambda: (0, 0)),
        ],
        out_specs=pl.BlockSpec((_B, _NC), lambda: (0, 0)),
        out_shape=jax.ShapeDtypeStruct((_B, _NC), jnp.float32),
    )(cls_rows, fw, fb, h1w_t, h1b, h2w_t, h2b)


def _moe_layer(li, hflat, n2w, n2b, rw, rb, w1, b1, w2, b2):
    x3p, gsel, slots2, te, tr = _route(li, hflat, n2w, n2b, rw, rb)
    slots_pad = slots2.reshape(_APAD)
    te = te.reshape(_NT)
    tr = tr.reshape(_NT)

    xdisp = _sc_dispatch(x3p, slots_pad)
    ybuf = _expert_ffn(li, te, tr, xdisp, w1, b1, w2, b2)
    gathered = _sc_collect(ybuf, slots_pad)
    return _combine(hflat, gsel, gathered)


def kernel(x, patch_w, patch_b, cls_token, pos_embed, norm1_w, norm1_b,
           attn_in_w, attn_in_b, attn_out_w, attn_out_b, norm2_w, norm2_b,
           router_w, router_b, e_w1, e_b1, e_w2, e_b2, fnorm_w, fnorm_b,
           head1_w, head1_b, head2_w, head2_b):
    x6 = x.reshape(_B, 3, _NPR, _P, _NPR, _P)
    pw = patch_w.transpose(1, 2, 3, 0)

    h = _patch_embed(x6, pw, patch_b.reshape(1, _D),
                     cls_token.reshape(1, _D), pos_embed.reshape(_S, _D))

    b1r = e_b1.reshape(_L, _E, 1, _HD)
    b2r = e_b2.reshape(_L, _E, 1, _D)
    n1w = norm1_w.reshape(_L, 1, _D)
    n1b = norm1_b.reshape(_L, 1, _D)
    bqkv = attn_in_b.reshape(_L, 1, 3 * _D)
    bout = attn_out_b.reshape(_L, 1, _D)
    n2w = norm2_w.reshape(_L, 1, _D)
    n2b = norm2_b.reshape(_L, 1, _D)
    rbr = router_b.reshape(_L, 1, _E)
    for i in range(_L):
        h = _attn_block(i, h, n1w, n1b, attn_in_w, bqkv, attn_out_w, bout)
        hflat = _moe_layer(i, h.reshape(_T, _D), n2w, n2b,
                           router_w, rbr, e_w1, b1r, e_w2, b2r)
        h = hflat.reshape(_B, _S, _D)

    cls_rows = h[:, 0, :]
    return _head(cls_rows, fnorm_w.reshape(1, _D), fnorm_b.reshape(1, _D),
                 head1_w, head1_b.reshape(1, _D),
                 head2_w, head2_b.reshape(1, _NC))


# speedup vs baseline: 1.2398x; 1.0184x over previous
import functools
import math

import jax
import jax.numpy as jnp
from jax import lax
from jax.experimental import pallas as pl
from jax.experimental.pallas import tpu as pltpu
from jax.experimental.pallas import tpu_sc as plsc

_L = 2
_D = 256
_E = 8
_HD = 1024
_H = 8
_P = 16
_IMG = 224
_B = 8
_S = (_IMG // _P) ** 2 + 1
_NC = 1000
_T = _B * _S
_DH = _D // _H

_TPAD = 1664
_CH = 104
_APAD = 2 * _TPAD
_NT = 32
_TB = 128
_DROWS = _NT * _TB + _TB
_TRASH = _NT * _TB
_NSC = 32


def _gelu(x):
    p0, p1, p2, p3 = (1.128381536146987, 0.1532354653121249,
                      0.04332742502107988, 0.0007576158168315916)
    q1, q2, q3 = (0.4691895897377689, 0.09460805147350791,
                  0.00935864267969743)
    ax = jnp.abs(x)
    z = jnp.minimum(ax * (1.0 / math.sqrt(2.0)), 3.25)
    u = z * z
    pn = ((p3 * u + p2) * u + p1) * u + p0
    qn = ((q3 * u + q2) * u + q1) * u + 1.0
    erf_abs = z * pn / qn
    return 0.5 * x + 0.5 * ax * erf_abs


def _ln(x, w, b):
    m = jnp.mean(x, axis=-1, keepdims=True)
    c = x - m
    v = jnp.mean(c * c, axis=-1, keepdims=True)
    return c * jax.lax.rsqrt(v + 1e-5) * w + b


_NPR = _IMG // _P


def _patch_kernel(x_ref, w_ref, b_ref, cls_ref, pos_ref, out_ref, acc_s):
    c = pl.program_id(0)
    i = pl.program_id(1)
    xs = x_ref[:, 0, :, 0, :, :].reshape(_B * _NPR * _NPR, _P)
    contrib = jnp.dot(xs, w_ref[0, 0], preferred_element_type=jnp.float32)
    first = jnp.logical_and(c == 0, i == 0)

    @pl.when(first)
    def _init():
        acc_s[...] = contrib

    @pl.when(jnp.logical_not(first))
    def _acc():
        acc_s[...] += contrib

    @pl.when(jnp.logical_and(c == 2, i == _P - 1))
    def _fin():
        pe = (acc_s[...] + b_ref[...]).reshape(_B, _NPR * _NPR, _D)
        cls_row = cls_ref[...] + pos_ref[:1, :]
        for b in range(_B):
            out_ref[b, :1, :] = cls_row
            out_ref[b, 1:, :] = pe[b] + pos_ref[1:, :]


def _patch_embed(x6, w, b, cls, pos):
    return pl.pallas_call(
        _patch_kernel,
        grid=(3, _P),
        in_specs=[
            pl.BlockSpec((_B, 1, _NPR, 1, _NPR, _P),
                         lambda c, i: (0, c, 0, i, 0, 0)),
            pl.BlockSpec((1, 1, _P, _D), lambda c, i: (c, i, 0, 0)),
            pl.BlockSpec((1, _D), lambda c, i: (0, 0)),
            pl.BlockSpec((1, _D), lambda c, i: (0, 0)),
            pl.BlockSpec((_S, _D), lambda c, i: (0, 0)),
        ],
        out_specs=pl.BlockSpec((_B, _S, _D), lambda c, i: (0, 0, 0)),
        out_shape=jax.ShapeDtypeStruct((_B, _S, _D), jnp.float32),
        scratch_shapes=[pltpu.VMEM((_B * _NPR * _NPR, _D), jnp.float32)],
        compiler_params=pltpu.CompilerParams(
            dimension_semantics=("arbitrary", "arbitrary")),
    )(x6, w, b, cls, pos)


def _attn_kernel(h_ref, n1w_ref, n1b_ref, wqkv_ref, bqkv_ref, wout_ref,
                 bout_ref, out_ref):
    x = h_ref[0]
    x2 = _ln(x, n1w_ref[0], n1b_ref[0])
    qkv = lax.dot_general(x2, wqkv_ref[0], (((1,), (1,)), ((), ())),
                          preferred_element_type=jnp.float32) + bqkv_ref[0]
    scale = 1.0 / math.sqrt(float(_DH))
    outs = []
    for hh in range(_H):
        q = qkv[:, hh * _DH:(hh + 1) * _DH]
        k = qkv[:, _D + hh * _DH:_D + (hh + 1) * _DH]
        v = qkv[:, 2 * _D + hh * _DH:2 * _D + (hh + 1) * _DH]
        s = lax.dot_general(q, k, (((1,), (1,)), ((), ())),
                            preferred_element_type=jnp.float32) * scale
        m = jnp.max(s, axis=-1, keepdims=True)
        p = jnp.exp(s - m)
        p = p * jax.lax.reciprocal(jnp.sum(p, axis=-1, keepdims=True))
        outs.append(jnp.dot(p, v, preferred_element_type=jnp.float32))
    o = jnp.concatenate(outs, axis=1)
    proj = lax.dot_general(o, wout_ref[0], (((1,), (1,)), ((), ())),
                           preferred_element_type=jnp.float32) + bout_ref[0]
    out_ref[0] = x + proj


def _attn_combine_kernel(h_ref, g_ref, ya_ref, yb_ref, n1w_ref, n1b_ref,
                         wqkv_ref, bqkv_ref, wout_ref, bout_ref, out_ref):
    g1 = g_ref[0, :, 0:1]
    g2 = g_ref[0, :, 1:2]
    x = h_ref[0] + g1 * ya_ref[0] + g2 * yb_ref[0]
    x2 = _ln(x, n1w_ref[0], n1b_ref[0])
    qkv = lax.dot_general(x2, wqkv_ref[0], (((1,), (1,)), ((), ())),
                          preferred_element_type=jnp.float32) + bqkv_ref[0]
    scale = 1.0 / math.sqrt(float(_DH))
    outs = []
    for hh in range(_H):
        q = qkv[:, hh * _DH:(hh + 1) * _DH]
        k = qkv[:, _D + hh * _DH:_D + (hh + 1) * _DH]
        v = qkv[:, 2 * _D + hh * _DH:2 * _D + (hh + 1) * _DH]
        s = lax.dot_general(q, k, (((1,), (1,)), ((), ())),
                            preferred_element_type=jnp.float32) * scale
        m = jnp.max(s, axis=-1, keepdims=True)
        p = jnp.exp(s - m)
        p = p * jax.lax.reciprocal(jnp.sum(p, axis=-1, keepdims=True))
        outs.append(jnp.dot(p, v, preferred_element_type=jnp.float32))
    o = jnp.concatenate(outs, axis=1)
    proj = lax.dot_general(o, wout_ref[0], (((1,), (1,)), ((), ())),
                           preferred_element_type=jnp.float32) + bout_ref[0]
    out_ref[0] = x + proj


def _attn_combine_block(li, hflat, gsel, gathered, n1w, n1b, wqkv, bqkv,
                        wout, bout):
    h3 = hflat.reshape(_B, _S, _D)
    g3 = gsel.reshape(_B, _S, _E)
    ya3 = gathered[:_T].reshape(_B, _S, _D)
    yb3 = gathered[_TPAD:_TPAD + _T].reshape(_B, _S, _D)
    return pl.pallas_call(
        _attn_combine_kernel,
        grid=(_B,),
        in_specs=[
            pl.BlockSpec((1, _S, _D), lambda i: (i, 0, 0)),
            pl.BlockSpec((1, _S, _E), lambda i: (i, 0, 0)),
            pl.BlockSpec((1, _S, _D), lambda i: (i, 0, 0)),
            pl.BlockSpec((1, _S, _D), lambda i: (i, 0, 0)),
            pl.BlockSpec((1, 1, _D), lambda i, li=li: (li, 0, 0)),
            pl.BlockSpec((1, 1, _D), lambda i, li=li: (li, 0, 0)),
            pl.BlockSpec((1, 3 * _D, _D), lambda i, li=li: (li, 0, 0)),
            pl.BlockSpec((1, 1, 3 * _D), lambda i, li=li: (li, 0, 0)),
            pl.BlockSpec((1, _D, _D), lambda i, li=li: (li, 0, 0)),
            pl.BlockSpec((1, 1, _D), lambda i, li=li: (li, 0, 0)),
        ],
        out_specs=pl.BlockSpec((1, _S, _D), lambda i: (i, 0, 0)),
        out_shape=jax.ShapeDtypeStruct((_B, _S, _D), jnp.float32),
    )(h3, g3, ya3, yb3, n1w, n1b, wqkv, bqkv, wout, bout)


def _attn_block(li, h, n1w, n1b, wqkv, bqkv, wout, bout):
    return pl.pallas_call(
        _attn_kernel,
        grid=(_B,),
        in_specs=[
            pl.BlockSpec((1, _S, _D), lambda i: (i, 0, 0)),
            pl.BlockSpec((1, 1, _D), lambda i, li=li: (li, 0, 0)),
            pl.BlockSpec((1, 1, _D), lambda i, li=li: (li, 0, 0)),
            pl.BlockSpec((1, 3 * _D, _D), lambda i, li=li: (li, 0, 0)),
            pl.BlockSpec((1, 1, 3 * _D), lambda i, li=li: (li, 0, 0)),
            pl.BlockSpec((1, _D, _D), lambda i, li=li: (li, 0, 0)),
            pl.BlockSpec((1, 1, _D), lambda i, li=li: (li, 0, 0)),
        ],
        out_specs=pl.BlockSpec((1, _S, _D), lambda i: (i, 0, 0)),
        out_shape=jax.ShapeDtypeStruct((_B, _S, _D), jnp.float32),
    )(h, n1w, n1b, wqkv, bqkv, wout, bout)


def _cumsum_rows(x, n):
    s = 1
    while s < n:
        shifted = jnp.concatenate(
            [jnp.zeros((s, x.shape[1]), x.dtype), x[:n - s]], axis=0)
        x = x + shifted
        s *= 2
    return x


def _route_kernel(h_ref, n2w_ref, n2b_ref, rw_ref, rb_ref,
                  x3_ref, gates_ref, slots_ref, te_ref, tr_ref):
    x3 = _ln(h_ref[...], n2w_ref[0], n2b_ref[0])
    x3_ref[:_T, :] = x3
    x3_ref[_T:, :] = jnp.zeros((_TPAD - _T, _D), jnp.float32)

    logits = lax.dot_general(x3, rw_ref[0], (((1,), (1,)), ((), ())),
                             preferred_element_type=jnp.float32) + rb_ref[0]
    lm = jnp.max(logits, axis=-1, keepdims=True)
    pe_ = jnp.exp(logits - lm)
    probs = pe_ * jax.lax.reciprocal(jnp.sum(pe_, axis=-1, keepdims=True))

    idx = lax.broadcasted_iota(jnp.int32, (_T, _E), 1)
    m1 = jnp.max(probs, axis=-1, keepdims=True)
    e1 = jnp.min(jnp.where(probs == m1, idx, _E), axis=-1, keepdims=True)
    oh1 = (idx == e1).astype(jnp.float32)
    probs2 = jnp.where(idx == e1, -jnp.inf, probs)
    m2 = jnp.max(probs2, axis=-1, keepdims=True)
    e2 = jnp.min(jnp.where(probs2 == m2, idx, _E), axis=-1, keepdims=True)
    oh2 = (idx == e2).astype(jnp.float32)

    gsum = m1 + m2
    g1 = m1 / gsum
    g2 = m2 / gsum
    sel0 = (idx == 0).astype(jnp.float32)
    sel1 = (idx == 1).astype(jnp.float32)
    gates_ref[...] = g1 * sel0 + g2 * sel1

    cum1 = _cumsum_rows(oh1, _T)
    c1 = cum1[_T - 1:_T, :]
    pos1 = cum1 - oh1
    cum2 = _cumsum_rows(oh2, _T)
    c2 = cum2[_T - 1:_T, :]
    pos2 = cum2 - oh2 + c1
    counts = c1 + c2

    asz = jnp.floor((counts + float(_TB - 1)) * (1.0 / _TB)) * float(_TB)
    acc = asz
    s = 1
    while s < _E:
        acc = acc + jnp.concatenate(
            [jnp.zeros((1, s), jnp.float32), acc[:, :_E - s]], axis=1)
        s *= 2
    seg = acc - asz

    slot1 = jnp.sum((pos1 + seg) * oh1, axis=-1, keepdims=True)
    slot2 = jnp.sum((pos2 + seg) * oh2, axis=-1, keepdims=True)

    trash = jnp.full((1, _TPAD - _T), float(_TRASH), jnp.float32)
    r1 = jnp.concatenate([slot1.reshape(1, _T), trash], axis=1)
    r2 = jnp.concatenate([slot2.reshape(1, _T), trash], axis=1)
    slots_ref[...] = jnp.concatenate([r1, r2], axis=0).astype(jnp.int32)

    ntiles = jnp.floor((counts + float(_TB - 1)) * (1.0 / _TB))
    cum = ntiles
    s = 1
    while s < _E:
        cum = cum + jnp.concatenate(
            [jnp.zeros((1, s), jnp.float32), cum[:, :_E - s]], axis=1)
        s *= 2
    lane8 = lax.broadcasted_iota(jnp.int32, (1, _E), 1)
    total = jnp.sum(jnp.where(lane8 == _E - 1, cum, 0.0), axis=-1,
                    keepdims=True)
    jj = lax.broadcasted_iota(jnp.int32, (_NT, _E), 0).astype(jnp.float32)
    te_raw = jnp.sum((jj >= cum).astype(jnp.float32), axis=-1,
                     keepdims=True)
    jcol = lax.broadcasted_iota(jnp.int32, (_NT, 1), 0).astype(jnp.float32)
    last = jnp.maximum(total - 1.0, 0.0)
    te_last = jnp.sum(jnp.where(jcol == last, te_raw, 0.0), axis=0,
                      keepdims=True)
    live = jcol < total
    te_ref[...] = jnp.where(live, te_raw, te_last).astype(jnp.int32)
    tr_ref[...] = jnp.where(live, jcol, last).astype(jnp.int32)


def _route(li, hflat, n2w, n2b, rw, rb):
    return pl.pallas_call(
        _route_kernel,
        grid=(1,),
        in_specs=[
            pl.BlockSpec((_T, _D), lambda g: (0, 0)),
            pl.BlockSpec((1, 1, _D), lambda g, li=li: (li, 0, 0)),
            pl.BlockSpec((1, 1, _D), lambda g, li=li: (li, 0, 0)),
            pl.BlockSpec((1, _E, _D), lambda g, li=li: (li, 0, 0)),
            pl.BlockSpec((1, 1, _E), lambda g, li=li: (li, 0, 0)),
        ],
        out_specs=[
            pl.BlockSpec((_TPAD, _D), lambda g: (0, 0)),
            pl.BlockSpec((_T, _E), lambda g: (0, 0)),
            pl.BlockSpec((2, _TPAD), lambda g: (0, 0)),
            pl.BlockSpec((_NT, 1), lambda g: (0, 0)),
            pl.BlockSpec((_NT, 1), lambda g: (0, 0)),
        ],
        out_shape=[
            jax.ShapeDtypeStruct((_TPAD, _D), jnp.float32),
            jax.ShapeDtypeStruct((_T, _E), jnp.float32),
            jax.ShapeDtypeStruct((2, _TPAD), jnp.int32),
            jax.ShapeDtypeStruct((_NT, 1), jnp.int32),
            jax.ShapeDtypeStruct((_NT, 1), jnp.int32),
        ],
    )(hflat, n2w, n2b, rw, rb)


def _sc_mesh():
    return plsc.VectorSubcoreMesh(core_axis_name="c", subcore_axis_name="s")


def _sc_dispatch(x3p, slots_pad):
    @functools.partial(
        pl.kernel,
        out_type=jax.ShapeDtypeStruct((_DROWS, _D), jnp.float32),
        mesh=_sc_mesh(),
        scratch_types=[
            pltpu.VMEM((_CH,), jnp.int32),
            pltpu.VMEM((_CH, _D), jnp.float32),
            pltpu.SemaphoreType.DMA,
        ],
    )
    def k(x3_hbm, slots_hbm, out_hbm, idx_v, rows_v, sem):
        w = lax.axis_index("s") * 2 + lax.axis_index("c")
        base = w * _CH
        t0 = (w % 16) * _CH
        pltpu.sync_copy(slots_hbm.at[pl.ds(base, _CH)], idx_v)
        pltpu.sync_copy(x3_hbm.at[pl.ds(t0, _CH)], rows_v)
        pltpu.async_copy(rows_v, out_hbm.at[idx_v], sem).wait()

    return k(x3p, slots_pad)


def _sc_collect(ybuf, slots_pad):
    @functools.partial(
        pl.kernel,
        out_type=jax.ShapeDtypeStruct((_APAD, _D), jnp.float32),
        mesh=_sc_mesh(),
        scratch_types=[
            pltpu.VMEM((_CH,), jnp.int32),
            pltpu.VMEM((_CH, _D), jnp.float32),
            pltpu.SemaphoreType.DMA,
        ],
    )
    def k(ybuf_hbm, slots_hbm, out_hbm, idx_v, rows_v, sem):
        w = lax.axis_index("s") * 2 + lax.axis_index("c")
        base = w * _CH
        pltpu.sync_copy(slots_hbm.at[pl.ds(base, _CH)], idx_v)
        pltpu.async_copy(ybuf_hbm.at[idx_v], rows_v, sem).wait()
        pltpu.sync_copy(rows_v, out_hbm.at[pl.ds(base, _CH)])

    return k(ybuf, slots_pad)


def _expert_kernel(te_ref, tr_ref, x_ref, w1_ref, b1_ref, w2_ref, b2_ref,
                   out_ref):
    del te_ref, tr_ref
    h1 = _gelu(jnp.dot(x_ref[...], w1_ref[0, 0],
                       preferred_element_type=jnp.float32) + b1_ref[0, 0])
    out_ref[...] = jnp.dot(h1, w2_ref[0, 0],
                           preferred_element_type=jnp.float32) + b2_ref[0, 0]


def _expert_ffn(li, te, tr, xdisp, w1, b1, w2, b2):
    spec = pltpu.PrefetchScalarGridSpec(
        num_scalar_prefetch=2,
        grid=(_NT,),
        in_specs=[
            pl.BlockSpec((_TB, _D), lambda j, te, tr: (tr[j], 0)),
            pl.BlockSpec((1, 1, _D, _HD),
                         lambda j, te, tr, li=li: (li, te[j], 0, 0)),
            pl.BlockSpec((1, 1, 1, _HD),
                         lambda j, te, tr, li=li: (li, te[j], 0, 0)),
            pl.BlockSpec((1, 1, _HD, _D),
                         lambda j, te, tr, li=li: (li, te[j], 0, 0)),
            pl.BlockSpec((1, 1, 1, _D),
                         lambda j, te, tr, li=li: (li, te[j], 0, 0)),
        ],
        out_specs=pl.BlockSpec((_TB, _D), lambda j, te, tr: (tr[j], 0)),
    )
    return pl.pallas_call(
        _expert_kernel,
        grid_spec=spec,
        out_shape=jax.ShapeDtypeStruct((_DROWS, _D), jnp.float32),
        compiler_params=pltpu.CompilerParams(
            dimension_semantics=("arbitrary",)),
    )(te, tr, xdisp, w1, b1, w2, b2)


def _combine_kernel(h_ref, g_ref, y_ref, out_ref):
    g1 = jnp.sum(g_ref[...] *
                 (lax.broadcasted_iota(jnp.int32, (_T, _E), 1) == 0),
                 axis=-1, keepdims=True)
    g2 = jnp.sum(g_ref[...] *
                 (lax.broadcasted_iota(jnp.int32, (_T, _E), 1) == 1),
                 axis=-1, keepdims=True)
    out_ref[...] = (h_ref[...] + g1 * y_ref[0, :_T, :]
                    + g2 * y_ref[1, :_T, :])


def _combine(hflat, gsel, gathered):
    return pl.pallas_call(
        _combine_kernel,
        in_specs=[
            pl.BlockSpec((_T, _D), lambda: (0, 0)),
            pl.BlockSpec((_T, _E), lambda: (0, 0)),
            pl.BlockSpec((2, _TPAD, _D), lambda: (0, 0, 0)),
        ],
        out_specs=pl.BlockSpec((_T, _D), lambda: (0, 0)),
        out_shape=jax.ShapeDtypeStruct((_T, _D), jnp.float32),
    )(hflat, gsel, gathered.reshape(2, _TPAD, _D))


def _head_kernel(h_ref, g_ref, y_ref, fw_ref, fb_ref, w1_ref, b1_ref,
                 w2_ref, b2_ref, out_ref):
    rows = []
    for b in range(_B):
        t = b * _S
        g1 = g_ref[t:t + 1, 0:1]
        g2 = g_ref[t:t + 1, 1:2]
        rows.append(h_ref[t:t + 1, :] + g1 * y_ref[0, t:t + 1, :]
                    + g2 * y_ref[1, t:t + 1, :])
    c = _ln(jnp.concatenate(rows, axis=0), fw_ref[...], fb_ref[...])
    z = _gelu(lax.dot_general(c, w1_ref[...], (((1,), (1,)), ((), ())),
                              preferred_element_type=jnp.float32)
              + b1_ref[...])
    out_ref[...] = lax.dot_general(z, w2_ref[...], (((1,), (1,)), ((), ())),
                                   preferred_element_type=jnp.float32
                                   ) + b2_ref[...]


def _head(hflat, gsel, gathered, fw, fb, h1w, h1b, h2w, h2b):
    return pl.pallas_call(
        _head_kernel,
        in_specs=[
            pl.BlockSpec((_T, _D), lambda: (0, 0)),
            pl.BlockSpec((_T, _E), lambda: (0, 0)),
            pl.BlockSpec((2, _TPAD, _D), lambda: (0, 0, 0)),
            pl.BlockSpec((1, _D), lambda: (0, 0)),
            pl.BlockSpec((1, _D), lambda: (0, 0)),
            pl.BlockSpec((_D, _D), lambda: (0, 0)),
            pl.BlockSpec((1, _D), lambda: (0, 0)),
            pl.BlockSpec((_NC, _D), lambda: (0, 0)),
            pl.BlockSpec((1, _NC), lambda: (0, 0)),
        ],
        out_specs=pl.BlockSpec((_B, _NC), lambda: (0, 0)),
        out_shape=jax.ShapeDtypeStruct((_B, _NC), jnp.float32),
    )(hflat, gsel, gathered.reshape(2, _TPAD, _D), fw, fb, h1w, h1b, h2w,
      h2b)


def _moe_layer(li, hflat, n2w, n2b, rw, rb, w1, b1, w2, b2):
    x3p, gsel, slots2, te, tr = _route(li, hflat, n2w, n2b, rw, rb)
    slots_pad = slots2.reshape(_APAD)
    te = te.reshape(_NT)
    tr = tr.reshape(_NT)

    xdisp = _sc_dispatch(x3p, slots_pad)
    ybuf = _expert_ffn(li, te, tr, xdisp, w1, b1, w2, b2)
    gathered = _sc_collect(ybuf, slots_pad)
    return gsel, gathered


def kernel(x, patch_w, patch_b, cls_token, pos_embed, norm1_w, norm1_b,
           attn_in_w, attn_in_b, attn_out_w, attn_out_b, norm2_w, norm2_b,
           router_w, router_b, e_w1, e_b1, e_w2, e_b2, fnorm_w, fnorm_b,
           head1_w, head1_b, head2_w, head2_b):
    x6 = x.reshape(_B, 3, _NPR, _P, _NPR, _P)
    pw = patch_w.transpose(1, 2, 3, 0)

    h = _patch_embed(x6, pw, patch_b.reshape(1, _D),
                     cls_token.reshape(1, _D), pos_embed.reshape(_S, _D))

    b1r = e_b1.reshape(_L, _E, 1, _HD)
    b2r = e_b2.reshape(_L, _E, 1, _D)
    n1w = norm1_w.reshape(_L, 1, _D)
    n1b = norm1_b.reshape(_L, 1, _D)
    bqkv = attn_in_b.reshape(_L, 1, 3 * _D)
    bout = attn_out_b.reshape(_L, 1, _D)
    n2w = norm2_w.reshape(_L, 1, _D)
    n2b = norm2_b.reshape(_L, 1, _D)
    rbr = router_b.reshape(_L, 1, _E)
    h = _attn_block(0, h, n1w, n1b, attn_in_w, bqkv, attn_out_w, bout)
    hflat0 = h.reshape(_T, _D)
    gsel0, gath0 = _moe_layer(0, hflat0, n2w, n2b, router_w, rbr,
                              e_w1, b1r, e_w2, b2r)
    h = _attn_combine_block(1, hflat0, gsel0, gath0, n1w, n1b, attn_in_w,
                            bqkv, attn_out_w, bout)
    hflat1 = h.reshape(_T, _D)
    gsel1, gath1 = _moe_layer(1, hflat1, n2w, n2b, router_w, rbr,
                              e_w1, b1r, e_w2, b2r)

    return _head(hflat1, gsel1, gath1,
                 fnorm_w.reshape(1, _D), fnorm_b.reshape(1, _D),
                 head1_w, head1_b.reshape(1, _D),
                 head2_w, head2_b.reshape(1, _NC))
